# baseline (speedup 1.0000x reference)
import jax
import jax.numpy as jnp
from jax.experimental import pallas as pl
from jax.experimental.pallas import tpu as pltpu


_BM = 200


def _mm_kernel(f_ref, a_ref, o_ref, f16_scr):
    @pl.when(pl.program_id(0) == 0)
    def _():
        f16_scr[...] = f_ref[...].astype(jnp.bfloat16)

    a16 = a_ref[...].astype(jnp.bfloat16)
    o_ref[...] = jnp.dot(a16, f16_scr[...], preferred_element_type=jnp.float32)


def kernel(features, A):
    m, k = A.shape
    d = features.shape[1]
    return pl.pallas_call(
        _mm_kernel,
        grid=(m // _BM,),
        in_specs=[
            pl.BlockSpec((k, d), lambda i: (0, 0)),
            pl.BlockSpec((_BM, k), lambda i: (i, 0)),
        ],
        out_specs=pl.BlockSpec((_BM, d), lambda i: (i, 0)),
        out_shape=jax.ShapeDtypeStruct((m, d), jnp.float32),
        scratch_shapes=[pltpu.VMEM((k, d), jnp.bfloat16)],
    )(features, A)


# speedup vs baseline: 1.0013x; 1.0013x over previous
import jax
import jax.numpy as jnp
from jax.experimental import pallas as pl
from jax.experimental.pallas import tpu as pltpu


_BM = 200


def _mm_kernel(f_ref, a0_ref, a1_ref, o_ref, f16_scr):
    @pl.when(pl.program_id(0) == 0)
    def _():
        f16_scr[...] = f_ref[...].astype(jnp.bfloat16)

    f16 = f16_scr[...]
    o_ref[:_BM, :] = jnp.dot(
        a0_ref[...].astype(jnp.bfloat16), f16, preferred_element_type=jnp.float32)
    o_ref[_BM:, :] = jnp.dot(
        a1_ref[...].astype(jnp.bfloat16), f16, preferred_element_type=jnp.float32)


def kernel(features, A):
    m, k = A.shape
    d = features.shape[1]
    return pl.pallas_call(
        _mm_kernel,
        grid=(m // (2 * _BM),),
        in_specs=[
            pl.BlockSpec((k, d), lambda i: (0, 0)),
            pl.BlockSpec((_BM, k), lambda i: (2 * i, 0)),
            pl.BlockSpec((_BM, k), lambda i: (2 * i + 1, 0)),
        ],
        out_specs=pl.BlockSpec((2 * _BM, d), lambda i: (i, 0)),
        out_shape=jax.ShapeDtypeStruct((m, d), jnp.float32),
        scratch_shapes=[pltpu.VMEM((k, d), jnp.bfloat16)],
    )(features, A, A)


# speedup vs baseline: 1.0148x; 1.0135x over previous
import jax
import jax.numpy as jnp
from jax.experimental import pallas as pl
from jax.experimental.pallas import tpu as pltpu


_BM = 400


def _mm_kernel(f_ref, a_ref, o_ref, f16_scr):
    @pl.when(pl.program_id(0) == 0)
    def _():
        f16_scr[...] = f_ref[...].astype(jnp.bfloat16)

    o_ref[...] = jax.lax.dot_general(
        a_ref[...], f16_scr[...],
        (((1,), (0,)), ((), ())),
        precision=jax.lax.Precision.DEFAULT,
        preferred_element_type=jnp.float32,
    )


def kernel(features, A):
    m, k = A.shape
    d = features.shape[1]
    return pl.pallas_call(
        _mm_kernel,
        grid=(m // _BM,),
        in_specs=[
            pl.BlockSpec((k, d), lambda i: (0, 0)),
            pl.BlockSpec((_BM, k), lambda i: (i, 0)),
        ],
        out_specs=pl.BlockSpec((_BM, d), lambda i: (i, 0)),
        out_shape=jax.ShapeDtypeStruct((m, d), jnp.float32),
        scratch_shapes=[pltpu.VMEM((k, d), jnp.bfloat16)],
    )(features, A)
